# baseline (speedup 1.0000x reference)
import functools

import jax
import jax.numpy as jnp
from jax import lax
from jax.experimental import pallas as pl
from jax.experimental.pallas import tpu as pltpu
from jax.experimental.pallas import tpu_sc as plsc

N = 100000
E = 1600000
IN_DIM = 128
HID = 64
NUM_LAYERS = 3

NB = 1024
NBLK = 98
N_PAD = NB * NBLK
ACC = 100096
ROWS_PT = ACC // 32
ZR = 391

LANES = 128
G = 4
E_PAD = 1605632
EROWS = E_PAD // LANES
R_TILE = EROWS // 16
SUP = R_TILE // G
R_CORE_DEG = EROWS // 2
R_TILE_DEG = R_CORE_DEG // 16
SUP_DEG = R_TILE_DEG // G

_mesh = plsc.VectorSubcoreMesh(
    core_axis_name="c", subcore_axis_name="s", num_cores=2, num_subcores=16
)


def _zero_vmem(buf, nrows):
    def body(i, carry):
        buf[i, :] = jnp.zeros((16,), jnp.float32)
        return carry
    lax.fori_loop(0, nrows, body, 0)


def _zero_acc(acc, zbuf, sid):
    for k in range(ROWS_PT // ZR):
        pltpu.sync_copy(zbuf, acc.at[pl.ds(sid * ROWS_PT + k * ZR, ZR)])


def _seg_body(h4, src_r, dst_r, out, sidx, didx, rows, zbuf, acc,
              isem, gsem, ssem):
    cid = lax.axis_index("c")
    sid = lax.axis_index("s")
    _zero_vmem(zbuf, ZR)
    for cc in range(2):
        chunk = cid * 2 + cc
        col = chunk * 16
        offv = jnp.full((16,), chunk * N_PAD, jnp.int32)
        _zero_acc(acc, zbuf, sid)
        plsc.subcore_barrier()

        def offset_src(p):
            for g in range(G):
                for q16 in range(LANES // 16):
                    sl = pl.ds(q16 * 16, 16)
                    sidx[p, g, sl] = sidx[p, g, sl] + offv

        def sup_body(s, carry):
            p = lax.rem(s, 2)
            q = 1 - p
            r0 = sid * R_TILE + s * G
            pltpu.sync_copy(src_r.at[pl.ds(r0, G)], sidx.at[p])
            pltpu.sync_copy(dst_r.at[pl.ds(r0, G)], didx.at[p])
            offset_src(p)
            gathers = [
                pltpu.async_copy(h4.at[sidx.at[p, g]], rows.at[p, g], gsem)
                for g in range(G)
            ]
            for cp in gathers:
                cp.wait()

            @pl.when(s > 0)
            def _():
                for g in range(G):
                    pltpu.make_async_copy(
                        rows.at[q, g], acc.at[didx.at[q, g]], ssem).wait()

            for g in range(G):
                pltpu.async_copy(
                    rows.at[p, g], acc.at[didx.at[p, g]], ssem, add=True)
            return carry

        lax.fori_loop(0, SUP, sup_body, 0)
        for g in range(G):
            pltpu.make_async_copy(
                rows.at[1, g], acc.at[didx.at[1, g]], ssem).wait()
        plsc.subcore_barrier()
        pltpu.sync_copy(
            acc.at[pl.ds(sid * ROWS_PT, ROWS_PT)],
            out.at[chunk, pl.ds(sid * ROWS_PT, ROWS_PT)],
        )
        plsc.subcore_barrier()


_seg_call = functools.partial(
    pl.kernel,
    _seg_body,
    out_type=jax.ShapeDtypeStruct((4, ACC, 16), jnp.float32),
    mesh=_mesh,
    compiler_params=pltpu.CompilerParams(use_tc_tiling_on_sc=False),
    scratch_types=[
        pltpu.VMEM((2, G, LANES), jnp.int32),
        pltpu.VMEM((2, G, LANES), jnp.int32),
        pltpu.VMEM((2, G, LANES, 16), jnp.float32),
        pltpu.VMEM((ZR, 16), jnp.float32),
        pltpu.VMEM_SHARED((ACC, 16), jnp.float32),
        pltpu.SemaphoreType.DMA,
        pltpu.SemaphoreType.DMA,
        pltpu.SemaphoreType.DMA,
    ],
)()


def _deg_body(dst_r, out, didx, ones_b, zbuf, acc, ssem):
    cid = lax.axis_index("c")
    sid = lax.axis_index("s")
    _zero_vmem(zbuf, ZR)

    def ones_init(i, carry):
        ones_b[i, :] = jnp.ones((16,), jnp.float32)
        return carry
    lax.fori_loop(0, LANES, ones_init, 0)

    _zero_acc(acc, zbuf, sid)
    plsc.subcore_barrier()

    def sup(s, carry):
        r0 = cid * R_CORE_DEG + sid * R_TILE_DEG + s * G
        pltpu.sync_copy(dst_r.at[pl.ds(r0, G)], didx)
        scatters = [
            pltpu.async_copy(ones_b, acc.at[didx.at[g]], ssem, add=True)
            for g in range(G)
        ]
        for cp in scatters:
            cp.wait()
        return carry

    lax.fori_loop(0, SUP_DEG, sup, 0)
    plsc.subcore_barrier()
    pltpu.sync_copy(
        acc.at[pl.ds(sid * ROWS_PT, ROWS_PT)],
        out.at[cid, pl.ds(sid * ROWS_PT, ROWS_PT)],
    )


_deg_call = functools.partial(
    pl.kernel,
    _deg_body,
    out_type=jax.ShapeDtypeStruct((2, ACC, 16), jnp.float32),
    mesh=_mesh,
    compiler_params=pltpu.CompilerParams(use_tc_tiling_on_sc=False),
    scratch_types=[
        pltpu.VMEM((G, LANES), jnp.int32),
        pltpu.VMEM((LANES, 16), jnp.float32),
        pltpu.VMEM((ZR, 16), jnp.float32),
        pltpu.VMEM_SHARED((ACC, 16), jnp.float32),
        pltpu.SemaphoreType.DMA,
    ],
)()


def _k_rdeg(dp_ref, r_ref):
    deg = dp_ref[0, :, 0] + dp_ref[1, :, 0]
    r_ref[...] = 1.0 / jnp.maximum(deg, 1.0)


def _rdeg_call(degp):
    return pl.pallas_call(
        _k_rdeg,
        grid=(NBLK,),
        in_specs=[pl.BlockSpec((2, NB, 16), lambda i: (0, i, 0))],
        out_specs=pl.BlockSpec((NB,), lambda i: (i,)),
        out_shape=jax.ShapeDtypeStruct((N_PAD,), jnp.float32),
    )(degp)


def _k_in(x_ref, w_ref, b_ref, h_ref, h4_ref):
    y = jnp.dot(x_ref[...], w_ref[...], preferred_element_type=jnp.float32)
    y = y + b_ref[...][None, :]
    h_ref[...] = y
    for c in range(4):
        h4_ref[c] = y[:, c * 16:(c + 1) * 16]


def _in_call(x, w, b):
    return pl.pallas_call(
        _k_in,
        grid=(NBLK,),
        in_specs=[
            pl.BlockSpec((NB, IN_DIM), lambda i: (i, 0)),
            pl.BlockSpec((IN_DIM, HID), lambda i: (0, 0)),
            pl.BlockSpec((HID,), lambda i: (0,)),
        ],
        out_specs=[
            pl.BlockSpec((NB, HID), lambda i: (i, 0)),
            pl.BlockSpec((4, NB, 16), lambda i: (0, i, 0)),
        ],
        out_shape=[
            jax.ShapeDtypeStruct((N_PAD, HID), jnp.float32),
            jax.ShapeDtypeStruct((4, N_PAD, 16), jnp.float32),
        ],
    )(x, w, b)


def _k_layer(h_ref, a4_ref, rd_ref, ws_ref, bs_ref, wn_ref, bn_ref,
             ho_ref, ho4_ref):
    agg = jnp.concatenate([a4_ref[c] for c in range(4)], axis=-1)
    mean = agg * rd_ref[...][:, None]
    y = (jnp.dot(h_ref[...], ws_ref[...], preferred_element_type=jnp.float32)
         + jnp.dot(mean, wn_ref[...], preferred_element_type=jnp.float32)
         + (bs_ref[...] + bn_ref[...])[None, :])
    y = jnp.maximum(y, 0.0)
    ho_ref[...] = y
    for c in range(4):
        ho4_ref[c] = y[:, c * 16:(c + 1) * 16]


def _layer_call(h, agg, rdeg, ws, bs, wn, bn):
    return pl.pallas_call(
        _k_layer,
        grid=(NBLK,),
        in_specs=[
            pl.BlockSpec((NB, HID), lambda i: (i, 0)),
            pl.BlockSpec((4, NB, 16), lambda i: (0, i, 0)),
            pl.BlockSpec((NB,), lambda i: (i,)),
            pl.BlockSpec((HID, HID), lambda i: (0, 0)),
            pl.BlockSpec((HID,), lambda i: (0,)),
            pl.BlockSpec((HID, HID), lambda i: (0, 0)),
            pl.BlockSpec((HID,), lambda i: (0,)),
        ],
        out_specs=[
            pl.BlockSpec((NB, HID), lambda i: (i, 0)),
            pl.BlockSpec((4, NB, 16), lambda i: (0, i, 0)),
        ],
        out_shape=[
            jax.ShapeDtypeStruct((N_PAD, HID), jnp.float32),
            jax.ShapeDtypeStruct((4, N_PAD, 16), jnp.float32),
        ],
    )(h, agg, rdeg, ws, bs, wn, bn)


def _k_fin(h0_ref, h1_ref, h2_ref, h3_ref, wm1_ref, bm1_ref, wm2_ref,
           bm2_ref, o_ref):
    wm1 = wm1_ref[...]
    z = (jnp.dot(h0_ref[...], wm1[0:64], preferred_element_type=jnp.float32)
         + jnp.dot(h1_ref[...], wm1[64:128], preferred_element_type=jnp.float32)
         + jnp.dot(h2_ref[...], wm1[128:192], preferred_element_type=jnp.float32)
         + jnp.dot(h3_ref[...], wm1[192:256], preferred_element_type=jnp.float32)
         + bm1_ref[...][None, :])
    z = jnp.where(z > 0, z, 0.01 * z)
    z = jnp.dot(z, wm2_ref[...], preferred_element_type=jnp.float32)
    z = z + bm2_ref[...][None, :]
    z = z - jnp.max(z, axis=-1, keepdims=True)
    ez = jnp.exp(z)
    o_ref[...] = ez / jnp.sum(ez, axis=-1, keepdims=True)


def _fin_call(h0, h1, h2, h3, wm1, bm1, wm2, bm2):
    return pl.pallas_call(
        _k_fin,
        grid=(NBLK,),
        in_specs=[pl.BlockSpec((NB, HID), lambda i: (i, 0))] * 4 + [
            pl.BlockSpec((4 * HID, HID), lambda i: (0, 0)),
            pl.BlockSpec((HID,), lambda i: (0,)),
            pl.BlockSpec((HID, HID), lambda i: (0, 0)),
            pl.BlockSpec((HID,), lambda i: (0,)),
        ],
        out_specs=pl.BlockSpec((NB, HID), lambda i: (i, 0)),
        out_shape=jax.ShapeDtypeStruct((N, HID), jnp.float32),
    )(h0, h1, h2, h3, wm1, bm1, wm2, bm2)


def kernel(x, graph, W_in, b_in, W_self, b_self, W_neigh, b_neigh,
           W_m1, b_m1, W_m2, b_m2):
    src = graph[0].astype(jnp.int32)
    dst = graph[1].astype(jnp.int32)
    src_p = jnp.concatenate([src, jnp.zeros((E_PAD - E,), jnp.int32)])
    dst_p = jnp.concatenate([dst, jnp.full((E_PAD - E,), N, jnp.int32)])
    src_r = src_p.reshape(EROWS, LANES)
    dst_r = dst_p.reshape(EROWS, LANES)

    degp = _deg_call(dst_r)
    rdeg = _rdeg_call(degp)
    h, h4 = _in_call(x, W_in, b_in)
    hs = [h]
    for l in range(NUM_LAYERS):
        agg = _seg_call(h4.reshape(4 * N_PAD, 16), src_r, dst_r)
        h, h4 = _layer_call(h, agg, rdeg, W_self[l], b_self[l],
                            W_neigh[l], b_neigh[l])
        hs.append(h)
    return _fin_call(hs[0], hs[1], hs[2], hs[3], W_m1, b_m1, W_m2, b_m2)


# speedup vs baseline: 1.0910x; 1.0910x over previous
import functools

import jax
import jax.numpy as jnp
from jax import lax
from jax.experimental import pallas as pl
from jax.experimental.pallas import tpu as pltpu
from jax.experimental.pallas import tpu_sc as plsc

N = 100000
E = 1600000
IN_DIM = 128
HID = 64
NUM_LAYERS = 3

NB = 1024
NBLK = 98
N_PAD = NB * NBLK
ACC = 100096
ROWS_PT = ACC // 32
LANES = 128
G = 6
E_PAD = 1609728
EROWS = E_PAD // LANES
R_TILE = EROWS // 16
SUP = R_TILE // G
PAR_LAST = (SUP - 1) % 2
G_DEG = 3
R_CORE_DEG = EROWS // 2
R_TILE_DEG = R_CORE_DEG // 16
SUP_DEG = R_TILE_DEG // G_DEG

_mesh = plsc.VectorSubcoreMesh(
    core_axis_name="c", subcore_axis_name="s", num_cores=2, num_subcores=16
)


def _zero_acc(acc, zeros_h, sid):
    pltpu.sync_copy(zeros_h, acc.at[pl.ds(sid * ROWS_PT, ROWS_PT)])


def _seg_body(h4, src_r, dst_r, zeros_h, out, sidx, didx, rows, acc,
              gsem, ssem):
    cid = lax.axis_index("c")
    sid = lax.axis_index("s")
    for cc in range(2):
        chunk = cid * 2 + cc
        offv = jnp.full((16,), chunk * N_PAD, jnp.int32)
        _zero_acc(acc, zeros_h, sid)
        plsc.subcore_barrier()

        def offset_src(p):
            for g in range(G):
                for q16 in range(LANES // 16):
                    sl = pl.ds(q16 * 16, 16)
                    sidx[p, g, sl] = sidx[p, g, sl] + offv

        def sup_body(s, carry):
            p = lax.rem(s, 2)
            q = 1 - p
            r0 = sid * R_TILE + s * G
            pltpu.sync_copy(src_r.at[pl.ds(r0, G)], sidx.at[p])
            pltpu.sync_copy(dst_r.at[pl.ds(r0, G)], didx.at[p])
            offset_src(p)
            gathers = [
                pltpu.async_copy(h4.at[sidx.at[p, g]], rows.at[p, g], gsem)
                for g in range(G)
            ]
            for cp in gathers:
                cp.wait()

            @pl.when(s > 0)
            def _():
                for g in range(G):
                    pltpu.make_async_copy(
                        rows.at[q, g], acc.at[didx.at[q, g]], ssem).wait()

            for g in range(G):
                pltpu.async_copy(
                    rows.at[p, g], acc.at[didx.at[p, g]], ssem, add=True)
            return carry

        lax.fori_loop(0, SUP, sup_body, 0)
        for g in range(G):
            pltpu.make_async_copy(
                rows.at[PAR_LAST, g], acc.at[didx.at[PAR_LAST, g]],
                ssem).wait()
        plsc.subcore_barrier()
        pltpu.sync_copy(
            acc.at[pl.ds(sid * ROWS_PT, ROWS_PT)],
            out.at[chunk, pl.ds(sid * ROWS_PT, ROWS_PT)],
        )
        plsc.subcore_barrier()


_seg_call = functools.partial(
    pl.kernel,
    _seg_body,
    out_type=jax.ShapeDtypeStruct((4, ACC, 16), jnp.float32),
    mesh=_mesh,
    compiler_params=pltpu.CompilerParams(use_tc_tiling_on_sc=False),
    scratch_types=[
        pltpu.VMEM((2, G, LANES), jnp.int32),
        pltpu.VMEM((2, G, LANES), jnp.int32),
        pltpu.VMEM((2, G, LANES, 16), jnp.float32),
        pltpu.VMEM_SHARED((ACC, 16), jnp.float32),
        pltpu.SemaphoreType.DMA,
        pltpu.SemaphoreType.DMA,
    ],
)()


def _deg_body(dst_r, zeros_h, out, didx, ones_b, acc, ssem):
    cid = lax.axis_index("c")
    sid = lax.axis_index("s")

    def ones_init(i, carry):
        ones_b[i, :] = jnp.ones((16,), jnp.float32)
        return carry
    lax.fori_loop(0, LANES, ones_init, 0)

    _zero_acc(acc, zeros_h, sid)
    plsc.subcore_barrier()

    def sup(s, carry):
        r0 = cid * R_CORE_DEG + sid * R_TILE_DEG + s * G_DEG
        pltpu.sync_copy(dst_r.at[pl.ds(r0, G_DEG)], didx)
        scatters = [
            pltpu.async_copy(ones_b, acc.at[didx.at[g]], ssem, add=True)
            for g in range(G_DEG)
        ]
        for cp in scatters:
            cp.wait()
        return carry

    lax.fori_loop(0, SUP_DEG, sup, 0)
    plsc.subcore_barrier()
    pltpu.sync_copy(
        acc.at[pl.ds(sid * ROWS_PT, ROWS_PT)],
        out.at[cid, pl.ds(sid * ROWS_PT, ROWS_PT)],
    )


_deg_call = functools.partial(
    pl.kernel,
    _deg_body,
    out_type=jax.ShapeDtypeStruct((2, ACC, 16), jnp.float32),
    mesh=_mesh,
    compiler_params=pltpu.CompilerParams(use_tc_tiling_on_sc=False),
    scratch_types=[
        pltpu.VMEM((G_DEG, LANES), jnp.int32),
        pltpu.VMEM((LANES, 16), jnp.float32),
        pltpu.VMEM_SHARED((ACC, 16), jnp.float32),
        pltpu.SemaphoreType.DMA,
    ],
)()


def _k_rdeg(dp_ref, r_ref):
    deg = dp_ref[0, :, 0] + dp_ref[1, :, 0]
    r_ref[...] = 1.0 / jnp.maximum(deg, 1.0)


def _rdeg_call(degp):
    return pl.pallas_call(
        _k_rdeg,
        grid=(NBLK,),
        in_specs=[pl.BlockSpec((2, NB, 16), lambda i: (0, i, 0))],
        out_specs=pl.BlockSpec((NB,), lambda i: (i,)),
        out_shape=jax.ShapeDtypeStruct((N_PAD,), jnp.float32),
    )(degp)


def _k_in(x_ref, w_ref, b_ref, h_ref, h4_ref):
    y = jnp.dot(x_ref[...], w_ref[...], preferred_element_type=jnp.float32)
    y = y + b_ref[...][None, :]
    h_ref[...] = y
    for c in range(4):
        h4_ref[c] = y[:, c * 16:(c + 1) * 16]


def _in_call(x, w, b):
    return pl.pallas_call(
        _k_in,
        grid=(NBLK,),
        in_specs=[
            pl.BlockSpec((NB, IN_DIM), lambda i: (i, 0)),
            pl.BlockSpec((IN_DIM, HID), lambda i: (0, 0)),
            pl.BlockSpec((HID,), lambda i: (0,)),
        ],
        out_specs=[
            pl.BlockSpec((NB, HID), lambda i: (i, 0)),
            pl.BlockSpec((4, NB, 16), lambda i: (0, i, 0)),
        ],
        out_shape=[
            jax.ShapeDtypeStruct((N_PAD, HID), jnp.float32),
            jax.ShapeDtypeStruct((4, N_PAD, 16), jnp.float32),
        ],
    )(x, w, b)


def _k_layer(h_ref, a4_ref, rd_ref, ws_ref, bs_ref, wn_ref, bn_ref,
             ho_ref, ho4_ref):
    agg = jnp.concatenate([a4_ref[c] for c in range(4)], axis=-1)
    mean = agg * rd_ref[...][:, None]
    y = (jnp.dot(h_ref[...], ws_ref[...], preferred_element_type=jnp.float32)
         + jnp.dot(mean, wn_ref[...], preferred_element_type=jnp.float32)
         + (bs_ref[...] + bn_ref[...])[None, :])
    y = jnp.maximum(y, 0.0)
    ho_ref[...] = y
    for c in range(4):
        ho4_ref[c] = y[:, c * 16:(c + 1) * 16]


def _layer_call(h, agg, rdeg, ws, bs, wn, bn):
    return pl.pallas_call(
        _k_layer,
        grid=(NBLK,),
        in_specs=[
            pl.BlockSpec((NB, HID), lambda i: (i, 0)),
            pl.BlockSpec((4, NB, 16), lambda i: (0, i, 0)),
            pl.BlockSpec((NB,), lambda i: (i,)),
            pl.BlockSpec((HID, HID), lambda i: (0, 0)),
            pl.BlockSpec((HID,), lambda i: (0,)),
            pl.BlockSpec((HID, HID), lambda i: (0, 0)),
            pl.BlockSpec((HID,), lambda i: (0,)),
        ],
        out_specs=[
            pl.BlockSpec((NB, HID), lambda i: (i, 0)),
            pl.BlockSpec((4, NB, 16), lambda i: (0, i, 0)),
        ],
        out_shape=[
            jax.ShapeDtypeStruct((N_PAD, HID), jnp.float32),
            jax.ShapeDtypeStruct((4, N_PAD, 16), jnp.float32),
        ],
    )(h, agg, rdeg, ws, bs, wn, bn)


def _k_fin(h0_ref, h1_ref, h2_ref, h3_ref, wm1_ref, bm1_ref, wm2_ref,
           bm2_ref, o_ref):
    wm1 = wm1_ref[...]
    z = (jnp.dot(h0_ref[...], wm1[0:64], preferred_element_type=jnp.float32)
         + jnp.dot(h1_ref[...], wm1[64:128], preferred_element_type=jnp.float32)
         + jnp.dot(h2_ref[...], wm1[128:192], preferred_element_type=jnp.float32)
         + jnp.dot(h3_ref[...], wm1[192:256], preferred_element_type=jnp.float32)
         + bm1_ref[...][None, :])
    z = jnp.where(z > 0, z, 0.01 * z)
    z = jnp.dot(z, wm2_ref[...], preferred_element_type=jnp.float32)
    z = z + bm2_ref[...][None, :]
    z = z - jnp.max(z, axis=-1, keepdims=True)
    ez = jnp.exp(z)
    o_ref[...] = ez / jnp.sum(ez, axis=-1, keepdims=True)


def _fin_call(h0, h1, h2, h3, wm1, bm1, wm2, bm2):
    return pl.pallas_call(
        _k_fin,
        grid=(NBLK,),
        in_specs=[pl.BlockSpec((NB, HID), lambda i: (i, 0))] * 4 + [
            pl.BlockSpec((4 * HID, HID), lambda i: (0, 0)),
            pl.BlockSpec((HID,), lambda i: (0,)),
            pl.BlockSpec((HID, HID), lambda i: (0, 0)),
            pl.BlockSpec((HID,), lambda i: (0,)),
        ],
        out_specs=pl.BlockSpec((NB, HID), lambda i: (i, 0)),
        out_shape=jax.ShapeDtypeStruct((N, HID), jnp.float32),
    )(h0, h1, h2, h3, wm1, bm1, wm2, bm2)


def kernel(x, graph, W_in, b_in, W_self, b_self, W_neigh, b_neigh,
           W_m1, b_m1, W_m2, b_m2):
    src = graph[0].astype(jnp.int32)
    dst = graph[1].astype(jnp.int32)
    src_p = jnp.concatenate([src, jnp.zeros((E_PAD - E,), jnp.int32)])
    dst_p = jnp.concatenate([dst, jnp.full((E_PAD - E,), N, jnp.int32)])
    src_r = src_p.reshape(EROWS, LANES)
    dst_r = dst_p.reshape(EROWS, LANES)

    zeros_h = jnp.zeros((ROWS_PT, 16), jnp.float32)
    degp = _deg_call(dst_r, zeros_h)
    rdeg = _rdeg_call(degp)
    h, h4 = _in_call(x, W_in, b_in)
    hs = [h]
    for l in range(NUM_LAYERS):
        agg = _seg_call(h4.reshape(4 * N_PAD, 16), src_r, dst_r, zeros_h)
        h, h4 = _layer_call(h, agg, rdeg, W_self[l], b_self[l],
                            W_neigh[l], b_neigh[l])
        hs.append(h)
    return _fin_call(hs[0], hs[1], hs[2], hs[3], W_m1, b_m1, W_m2, b_m2)


# speedup vs baseline: 1.1737x; 1.0758x over previous
import functools

import jax
import jax.numpy as jnp
from jax import lax
from jax.experimental import pallas as pl
from jax.experimental.pallas import tpu as pltpu
from jax.experimental.pallas import tpu_sc as plsc

N = 100000
E = 1600000
IN_DIM = 128
HID = 64
NUM_LAYERS = 3

NB = 1024
NBLK = 98
N_PAD = NB * NBLK
ACC = 100096
ROWS_PT = ACC // 32
LANES = 128
G = 6
E_PAD = 1609728
EROWS = E_PAD // LANES
R_TILE = EROWS // 16
SUP = R_TILE // G
PAR_LAST = (SUP - 1) % 2
G_DEG = 3
R_CORE_DEG = EROWS // 2
R_TILE_DEG = R_CORE_DEG // 16
SUP_DEG = R_TILE_DEG // G_DEG

_mesh = plsc.VectorSubcoreMesh(
    core_axis_name="c", subcore_axis_name="s", num_cores=2, num_subcores=16
)


def _zero_acc(acc, zeros_h, sid):
    pltpu.sync_copy(zeros_h, acc.at[pl.ds(sid * ROWS_PT, ROWS_PT)])


def _seg_body(h4, esr, zeros_h, out, eidx, rows, acc, isem, gsem, ssem):
    cid = lax.axis_index("c")
    sid = lax.axis_index("s")
    for cc in range(2):
        chunk = cid * 2 + cc
        offv = jnp.full((16,), chunk * N_PAD, jnp.int32)
        _zero_acc(acc, zeros_h, sid)
        plsc.subcore_barrier()

        def idx_prefetch(s, p):
            r0 = sid * R_TILE + s * G
            return pltpu.async_copy(esr.at[pl.ds(r0, G)], eidx.at[p], isem)

        def offset_src(p):
            for g in range(G):
                for q16 in range(LANES // 16):
                    sl = pl.ds(q16 * 16, 16)
                    eidx[p, g, 0, sl] = eidx[p, g, 0, sl] + offv

        idx_prefetch(0, 0)

        def sup_body(s, carry):
            p = lax.rem(s, 2)
            q = 1 - p
            pltpu.make_async_copy(
                esr.at[pl.ds(0, G)], eidx.at[p], isem).wait()
            offset_src(p)
            gathers = [
                pltpu.async_copy(
                    h4.at[eidx.at[p, g, 0]], rows.at[p, g], gsem)
                for g in range(G)
            ]
            for cp in gathers:
                cp.wait()

            @pl.when(s > 0)
            def _():
                for g in range(G):
                    pltpu.make_async_copy(
                        rows.at[q, g], acc.at[eidx.at[q, g, 1]], ssem).wait()

            @pl.when(s + 1 < SUP)
            def _():
                idx_prefetch(s + 1, q)

            for g in range(G):
                pltpu.async_copy(
                    rows.at[p, g], acc.at[eidx.at[p, g, 1]], ssem, add=True)
            return carry

        lax.fori_loop(0, SUP, sup_body, 0)
        for g in range(G):
            pltpu.make_async_copy(
                rows.at[PAR_LAST, g], acc.at[eidx.at[PAR_LAST, g, 1]],
                ssem).wait()
        plsc.subcore_barrier()
        pltpu.sync_copy(
            acc.at[pl.ds(sid * ROWS_PT, ROWS_PT)],
            out.at[chunk, pl.ds(sid * ROWS_PT, ROWS_PT)],
        )
        plsc.subcore_barrier()


_seg_call = functools.partial(
    pl.kernel,
    _seg_body,
    out_type=jax.ShapeDtypeStruct((4, ACC, 16), jnp.float32),
    mesh=_mesh,
    compiler_params=pltpu.CompilerParams(use_tc_tiling_on_sc=False),
    scratch_types=[
        pltpu.VMEM((2, G, 2, LANES), jnp.int32),
        pltpu.VMEM((2, G, LANES, 16), jnp.float32),
        pltpu.VMEM_SHARED((ACC, 16), jnp.float32),
        pltpu.SemaphoreType.DMA,
        pltpu.SemaphoreType.DMA,
        pltpu.SemaphoreType.DMA,
    ],
)()


def _deg_body(dst_r, zeros_h, out, didx, ones_b, acc, ssem):
    cid = lax.axis_index("c")
    sid = lax.axis_index("s")

    def ones_init(i, carry):
        ones_b[i, :] = jnp.ones((16,), jnp.float32)
        return carry
    lax.fori_loop(0, LANES, ones_init, 0)

    _zero_acc(acc, zeros_h, sid)
    plsc.subcore_barrier()

    def sup(s, carry):
        r0 = cid * R_CORE_DEG + sid * R_TILE_DEG + s * G_DEG
        pltpu.sync_copy(dst_r.at[pl.ds(r0, G_DEG)], didx)
        scatters = [
            pltpu.async_copy(ones_b, acc.at[didx.at[g]], ssem, add=True)
            for g in range(G_DEG)
        ]
        for cp in scatters:
            cp.wait()
        return carry

    lax.fori_loop(0, SUP_DEG, sup, 0)
    plsc.subcore_barrier()
    pltpu.sync_copy(
        acc.at[pl.ds(sid * ROWS_PT, ROWS_PT)],
        out.at[cid, pl.ds(sid * ROWS_PT, ROWS_PT)],
    )


_deg_call = functools.partial(
    pl.kernel,
    _deg_body,
    out_type=jax.ShapeDtypeStruct((2, ACC, 16), jnp.float32),
    mesh=_mesh,
    compiler_params=pltpu.CompilerParams(use_tc_tiling_on_sc=False),
    scratch_types=[
        pltpu.VMEM((G_DEG, LANES), jnp.int32),
        pltpu.VMEM((LANES, 16), jnp.float32),
        pltpu.VMEM_SHARED((ACC, 16), jnp.float32),
        pltpu.SemaphoreType.DMA,
    ],
)()


def _k_rdeg(dp_ref, r_ref):
    deg = dp_ref[0, :, 0] + dp_ref[1, :, 0]
    r_ref[...] = 1.0 / jnp.maximum(deg, 1.0)


def _rdeg_call(degp):
    return pl.pallas_call(
        _k_rdeg,
        grid=(NBLK,),
        in_specs=[pl.BlockSpec((2, NB, 16), lambda i: (0, i, 0))],
        out_specs=pl.BlockSpec((NB,), lambda i: (i,)),
        out_shape=jax.ShapeDtypeStruct((N_PAD,), jnp.float32),
    )(degp)


def _k_in(x_ref, w_ref, b_ref, h_ref, h4_ref):
    y = jnp.dot(x_ref[...], w_ref[...], preferred_element_type=jnp.float32)
    y = y + b_ref[...][None, :]
    h_ref[...] = y
    for c in range(4):
        h4_ref[c] = y[:, c * 16:(c + 1) * 16]


def _in_call(x, w, b):
    return pl.pallas_call(
        _k_in,
        grid=(NBLK,),
        in_specs=[
            pl.BlockSpec((NB, IN_DIM), lambda i: (i, 0)),
            pl.BlockSpec((IN_DIM, HID), lambda i: (0, 0)),
            pl.BlockSpec((HID,), lambda i: (0,)),
        ],
        out_specs=[
            pl.BlockSpec((NB, HID), lambda i: (i, 0)),
            pl.BlockSpec((4, NB, 16), lambda i: (0, i, 0)),
        ],
        out_shape=[
            jax.ShapeDtypeStruct((N_PAD, HID), jnp.float32),
            jax.ShapeDtypeStruct((4, N_PAD, 16), jnp.float32),
        ],
    )(x, w, b)


def _k_layer(h_ref, a4_ref, rd_ref, ws_ref, bs_ref, wn_ref, bn_ref,
             ho_ref, ho4_ref):
    agg = jnp.concatenate([a4_ref[c] for c in range(4)], axis=-1)
    mean = agg * rd_ref[...][:, None]
    y = (jnp.dot(h_ref[...], ws_ref[...], preferred_element_type=jnp.float32)
         + jnp.dot(mean, wn_ref[...], preferred_element_type=jnp.float32)
         + (bs_ref[...] + bn_ref[...])[None, :])
    y = jnp.maximum(y, 0.0)
    ho_ref[...] = y
    for c in range(4):
        ho4_ref[c] = y[:, c * 16:(c + 1) * 16]


def _layer_call(h, agg, rdeg, ws, bs, wn, bn):
    return pl.pallas_call(
        _k_layer,
        grid=(NBLK,),
        in_specs=[
            pl.BlockSpec((NB, HID), lambda i: (i, 0)),
            pl.BlockSpec((4, NB, 16), lambda i: (0, i, 0)),
            pl.BlockSpec((NB,), lambda i: (i,)),
            pl.BlockSpec((HID, HID), lambda i: (0, 0)),
            pl.BlockSpec((HID,), lambda i: (0,)),
            pl.BlockSpec((HID, HID), lambda i: (0, 0)),
            pl.BlockSpec((HID,), lambda i: (0,)),
        ],
        out_specs=[
            pl.BlockSpec((NB, HID), lambda i: (i, 0)),
            pl.BlockSpec((4, NB, 16), lambda i: (0, i, 0)),
        ],
        out_shape=[
            jax.ShapeDtypeStruct((N_PAD, HID), jnp.float32),
            jax.ShapeDtypeStruct((4, N_PAD, 16), jnp.float32),
        ],
    )(h, agg, rdeg, ws, bs, wn, bn)


def _k_fin(h0_ref, h1_ref, h2_ref, h3_ref, wm1_ref, bm1_ref, wm2_ref,
           bm2_ref, o_ref):
    wm1 = wm1_ref[...]
    z = (jnp.dot(h0_ref[...], wm1[0:64], preferred_element_type=jnp.float32)
         + jnp.dot(h1_ref[...], wm1[64:128], preferred_element_type=jnp.float32)
         + jnp.dot(h2_ref[...], wm1[128:192], preferred_element_type=jnp.float32)
         + jnp.dot(h3_ref[...], wm1[192:256], preferred_element_type=jnp.float32)
         + bm1_ref[...][None, :])
    z = jnp.where(z > 0, z, 0.01 * z)
    z = jnp.dot(z, wm2_ref[...], preferred_element_type=jnp.float32)
    z = z + bm2_ref[...][None, :]
    z = z - jnp.max(z, axis=-1, keepdims=True)
    ez = jnp.exp(z)
    o_ref[...] = ez / jnp.sum(ez, axis=-1, keepdims=True)


def _fin_call(h0, h1, h2, h3, wm1, bm1, wm2, bm2):
    return pl.pallas_call(
        _k_fin,
        grid=(NBLK,),
        in_specs=[pl.BlockSpec((NB, HID), lambda i: (i, 0))] * 4 + [
            pl.BlockSpec((4 * HID, HID), lambda i: (0, 0)),
            pl.BlockSpec((HID,), lambda i: (0,)),
            pl.BlockSpec((HID, HID), lambda i: (0, 0)),
            pl.BlockSpec((HID,), lambda i: (0,)),
        ],
        out_specs=pl.BlockSpec((NB, HID), lambda i: (i, 0)),
        out_shape=jax.ShapeDtypeStruct((N, HID), jnp.float32),
    )(h0, h1, h2, h3, wm1, bm1, wm2, bm2)


def kernel(x, graph, W_in, b_in, W_self, b_self, W_neigh, b_neigh,
           W_m1, b_m1, W_m2, b_m2):
    src = graph[0].astype(jnp.int32)
    dst = graph[1].astype(jnp.int32)
    src_p = jnp.concatenate([src, jnp.zeros((E_PAD - E,), jnp.int32)])
    dst_p = jnp.concatenate([dst, jnp.full((E_PAD - E,), N, jnp.int32)])
    src_r = src_p.reshape(EROWS, LANES)
    dst_r = dst_p.reshape(EROWS, LANES)
    esr = jnp.stack([src_r, dst_r], axis=1)

    zeros_h = jnp.zeros((ROWS_PT, 16), jnp.float32)
    degp = _deg_call(dst_r, zeros_h)
    rdeg = _rdeg_call(degp)
    h, h4 = _in_call(x, W_in, b_in)
    hs = [h]
    for l in range(NUM_LAYERS):
        agg = _seg_call(h4.reshape(4 * N_PAD, 16), esr, zeros_h)
        h, h4 = _layer_call(h, agg, rdeg, W_self[l], b_self[l],
                            W_neigh[l], b_neigh[l])
        hs.append(h)
    return _fin_call(hs[0], hs[1], hs[2], hs[3], W_m1, b_m1, W_m2, b_m2)
